# baseline (speedup 1.0000x reference)
import functools

import jax
import jax.numpy as jnp
from jax import lax
from jax.experimental import pallas as pl
from jax.experimental.pallas import tpu as pltpu
from jax.experimental.pallas import tpu_sc as plsc

_B = 4096
_K = 512
_D = 64
_RB = 2048
_NB = _B // _RB


def _assign_body(x_ref, ct_ref, c_ref, out_ref,
                 c1_ref, c2_ref, c3_ref, t1_ref, t2_ref, cn_ref):
    @pl.when(pl.program_id(0) == 0)
    def _():
        c = c_ref[...]
        c1 = c.astype(jnp.bfloat16)
        r1 = c - c1.astype(jnp.float32)
        c2 = r1.astype(jnp.bfloat16)
        c3 = (r1 - c2.astype(jnp.float32)).astype(jnp.bfloat16)
        c1_ref[...] = c1
        c2_ref[...] = c2
        c3_ref[...] = c3
        ct0 = ct_ref[...]
        t1 = ct0.astype(jnp.bfloat16)
        t1_ref[...] = t1
        t2_ref[...] = (ct0 - t1.astype(jnp.float32)).astype(jnp.bfloat16)
        cn_ref[...] = jnp.sum(ct0 * ct0, axis=0, keepdims=True)

    x = x_ref[...]
    cn = cn_ref[...]
    x1 = x.astype(jnp.bfloat16)
    x2 = (x - x1.astype(jnp.float32)).astype(jnp.bfloat16)
    t1 = t1_ref[...]
    t2 = t2_ref[...]

    def bmm(a, b):
        return lax.dot_general(a, b, (((1,), (0,)), ((), ())),
                               preferred_element_type=jnp.float32)

    xc = bmm(x1, t1) + (bmm(x1, t2) + bmm(x2, t1))
    scores = cn - 2.0 * xc
    ids = lax.broadcasted_iota(jnp.int32, scores.shape, 1)

    m1 = jnp.min(scores, axis=1, keepdims=True)
    i1 = jnp.min(jnp.where(scores == m1, ids, _K), axis=1, keepdims=True)
    hit1 = ids == i1
    masked = jnp.where(hit1, jnp.inf, scores)
    m2 = jnp.min(masked, axis=1, keepdims=True)
    hit2 = masked == m2
    i2 = jnp.min(jnp.where(hit2, ids, _K), axis=1, keepdims=True)

    def pick_rows(oh_bool):
        oh = oh_bool.astype(jnp.bfloat16)
        parts = [
            lax.dot_general(oh, p_ref[...], (((1,), (0,)), ((), ())),
                            preferred_element_type=jnp.float32)
            for p_ref in (c1_ref, c2_ref, c3_ref)
        ]
        return (parts[0] + parts[1]) + parts[2]

    ca = pick_rows(hit1)
    cb = pick_rows(ids == i2)

    def tree8(s):
        a = s[:, 0:4] + s[:, 4:8]
        b = a[:, 0:2] + a[:, 2:4]
        return b[:, 0:1] + b[:, 1:2]

    def ref_dist(t):
        acc = tree8(t[:, 0:8])
        for v in range(1, 8):
            acc = acc + tree8(t[:, 8 * v:8 * v + 8])
        return acc * lax.rsqrt(acc)

    da_ = x - ca
    db_ = x - cb
    da = ref_dist(da_ * da_)
    db = ref_dist(db_ * db_)
    pick = jnp.where(db < da, i2, i1)
    pick = jnp.where(db == da, jnp.minimum(i1, i2), pick)
    out_ref[...] = pick


def _assignments(inputs, clusters_t, clusters):
    out = pl.pallas_call(
        _assign_body,
        grid=(_NB,),
        in_specs=[
            pl.BlockSpec((_RB, _D), lambda i: (i, 0)),
            pl.BlockSpec((_D, _K), lambda i: (0, 0)),
            pl.BlockSpec((_K, _D), lambda i: (0, 0)),
        ],
        out_specs=pl.BlockSpec((_RB, 1), lambda i: (i, 0)),
        out_shape=jax.ShapeDtypeStruct((_B, 1), jnp.int32),
        scratch_shapes=[
            pltpu.VMEM((_K, _D), jnp.bfloat16),
            pltpu.VMEM((_K, _D), jnp.bfloat16),
            pltpu.VMEM((_K, _D), jnp.bfloat16),
            pltpu.VMEM((_D, _K), jnp.bfloat16),
            pltpu.VMEM((_D, _K), jnp.bfloat16),
            pltpu.VMEM((1, _K), jnp.float32),
        ],
        compiler_params=pltpu.CompilerParams(
            allow_input_fusion=[False, True, False]),
    )(inputs, clusters_t, clusters)
    return out.reshape(_B)


_NC = 1
_NS = 16
_NW = _NC * _NS
_BPW = _B // _NW


@functools.cache
def _gather_rows():
    @functools.partial(
        pl.kernel,
        mesh=plsc.VectorSubcoreMesh(core_axis_name="c", subcore_axis_name="s",
                                    num_cores=_NC),
        out_type=jax.ShapeDtypeStruct((_B, _D), jnp.float32),
        scratch_types=[
            pltpu.VMEM((_BPW,), jnp.int32),
            pltpu.VMEM((_BPW, _D), jnp.float32),
            pltpu.SemaphoreType.DMA,
        ],
        compiler_params=pltpu.CompilerParams(use_tc_tiling_on_sc=False),
    )
    def gather_k(table_hbm, idx_hbm, out_hbm, idx_v, rows_v, sem):
        wid = lax.axis_index("s") * _NC + lax.axis_index("c")
        base = wid * _BPW
        pltpu.sync_copy(idx_hbm.at[pl.ds(base, _BPW)], idx_v)
        pltpu.async_copy(table_hbm.at[idx_v], rows_v, sem).wait()
        pltpu.sync_copy(rows_v, out_hbm.at[pl.ds(base, _BPW)])

    return gather_k


def kernel(inputs, clusters):
    assignments = _assignments(inputs, clusters.T, clusters)
    return _gather_rows()(clusters, assignments)


# speedup vs baseline: 1.2788x; 1.2788x over previous
import functools

import jax
import jax.numpy as jnp
from jax import lax
from jax.experimental import pallas as pl
from jax.experimental.pallas import tpu as pltpu
from jax.experimental.pallas import tpu_sc as plsc

_B = 4096
_K = 512
_D = 64
_RB = 2048
_NB = _B // _RB


def _assign_body(x_ref, ct_ref, c_ref, out_ref,
                 c1_ref, c2_ref, c3_ref, t1_ref, t2_ref, cn_ref):
    @pl.when(pl.program_id(0) == 0)
    def _():
        c = c_ref[...]
        c1 = c.astype(jnp.bfloat16)
        r1 = c - c1.astype(jnp.float32)
        c2 = r1.astype(jnp.bfloat16)
        c3 = (r1 - c2.astype(jnp.float32)).astype(jnp.bfloat16)
        c1_ref[...] = c1
        c2_ref[...] = c2
        c3_ref[...] = c3
        ct0 = ct_ref[...]
        t1 = ct0.astype(jnp.bfloat16)
        t1_ref[...] = t1
        t2_ref[...] = (ct0 - t1.astype(jnp.float32)).astype(jnp.bfloat16)
        cn_ref[...] = jnp.sum(ct0 * ct0, axis=0, keepdims=True)

    x = x_ref[...]
    cn = cn_ref[...]
    x1 = x.astype(jnp.bfloat16)
    x2 = (x - x1.astype(jnp.float32)).astype(jnp.bfloat16)
    t1 = t1_ref[...]
    t2 = t2_ref[...]

    def bmm(a, b):
        return lax.dot_general(a, b, (((1,), (0,)), ((), ())),
                               preferred_element_type=jnp.float32)

    xc = bmm(x1, t1) + (bmm(x1, t2) + bmm(x2, t1))
    scores = cn - 2.0 * xc
    ids = lax.broadcasted_iota(jnp.int32, scores.shape, 1)

    m1 = jnp.min(scores, axis=1, keepdims=True)
    i1 = jnp.min(jnp.where(scores == m1, ids, _K), axis=1, keepdims=True)
    hit1 = ids == i1
    masked = jnp.where(hit1, jnp.inf, scores)
    m2 = jnp.min(masked, axis=1, keepdims=True)
    hit2 = masked == m2
    i2 = jnp.min(jnp.where(hit2, ids, _K), axis=1, keepdims=True)

    def pick_rows(oh_bool):
        oh = oh_bool.astype(jnp.bfloat16)
        parts = [
            lax.dot_general(oh, p_ref[...], (((1,), (0,)), ((), ())),
                            preferred_element_type=jnp.float32)
            for p_ref in (c1_ref, c2_ref, c3_ref)
        ]
        return (parts[0] + parts[1]) + parts[2]

    ca = pick_rows(hit1)
    cb = pick_rows(ids == i2)

    def ref_dist(diff_t):
        t3 = (diff_t * diff_t).reshape(8, 8, _RB)
        a = t3[:, 0:4, :] + t3[:, 4:8, :]
        b = a[:, 0:2, :] + a[:, 2:4, :]
        c = b[:, 0:1, :] + b[:, 1:2, :]
        acc = c[0]
        for v in range(1, 8):
            acc = acc + c[v]
        return acc * lax.rsqrt(acc)

    da = ref_dist(jnp.transpose(x - ca))
    db = ref_dist(jnp.transpose(x - cb))
    i1t = jnp.transpose(i1)
    i2t = jnp.transpose(i2)
    pick = jnp.where(db < da, i2t, i1t)
    pick = jnp.where(db == da, jnp.minimum(i1t, i2t), pick)
    out_ref[0] = pick


def _assignments(inputs, clusters_t, clusters):
    out = pl.pallas_call(
        _assign_body,
        grid=(_NB,),
        in_specs=[
            pl.BlockSpec((_RB, _D), lambda i: (i, 0)),
            pl.BlockSpec((_D, _K), lambda i: (0, 0)),
            pl.BlockSpec((_K, _D), lambda i: (0, 0)),
        ],
        out_specs=pl.BlockSpec((1, 1, _RB), lambda i: (i, 0, 0)),
        out_shape=jax.ShapeDtypeStruct((_NB, 1, _RB), jnp.int32),
        scratch_shapes=[
            pltpu.VMEM((_K, _D), jnp.bfloat16),
            pltpu.VMEM((_K, _D), jnp.bfloat16),
            pltpu.VMEM((_K, _D), jnp.bfloat16),
            pltpu.VMEM((_D, _K), jnp.bfloat16),
            pltpu.VMEM((_D, _K), jnp.bfloat16),
            pltpu.VMEM((1, _K), jnp.float32),
        ],
        compiler_params=pltpu.CompilerParams(
            allow_input_fusion=[False, True, False]),
    )(inputs, clusters_t, clusters)
    return out.reshape(_B)


_NC = 1
_NS = 16
_NW = _NC * _NS
_BPW = _B // _NW


@functools.cache
def _gather_rows():
    @functools.partial(
        pl.kernel,
        mesh=plsc.VectorSubcoreMesh(core_axis_name="c", subcore_axis_name="s",
                                    num_cores=_NC),
        out_type=jax.ShapeDtypeStruct((_B, _D), jnp.float32),
        scratch_types=[
            pltpu.VMEM((_BPW,), jnp.int32),
            pltpu.VMEM((_BPW, _D), jnp.float32),
            pltpu.SemaphoreType.DMA,
        ],
        compiler_params=pltpu.CompilerParams(use_tc_tiling_on_sc=False),
    )
    def gather_k(table_hbm, idx_hbm, out_hbm, idx_v, rows_v, sem):
        wid = lax.axis_index("s") * _NC + lax.axis_index("c")
        base = wid * _BPW
        pltpu.sync_copy(idx_hbm.at[pl.ds(base, _BPW)], idx_v)
        pltpu.async_copy(table_hbm.at[idx_v], rows_v, sem).wait()
        pltpu.sync_copy(rows_v, out_hbm.at[pl.ds(base, _BPW)])

    return gather_k


def kernel(inputs, clusters):
    assignments = _assignments(inputs, clusters.T, clusters)
    return _gather_rows()(clusters, assignments)
